# baseline (speedup 1.0000x reference)
import functools

import jax
import jax.numpy as jnp
from jax import lax
from jax.experimental import pallas as pl
from jax.experimental.pallas import tpu as pltpu
from jax.experimental.pallas import tpu_sc as plsc

_D_MODEL = 768
_NUM_EXPERTS = 8
_TOP_K = 2
_BLOCK_ROWS = 4096

_NUM_SC = 2
_NUM_SUBCORES = 16
_NW = _NUM_SC * _NUM_SUBCORES
_LANES = 16


def _matmul_body(x_ref, wt_ref, logits_ref):
    x_blk = x_ref[...]
    wt = wt_ref[...]
    logits = jnp.dot(x_blk, wt, preferred_element_type=jnp.float32)
    logits_ref[...] = logits.T


def _gate_logits(xf, wt, n):
    return pl.pallas_call(
        _matmul_body,
        grid=(n // _BLOCK_ROWS,),
        in_specs=[
            pl.BlockSpec((_BLOCK_ROWS, _D_MODEL), lambda i: (i, 0)),
            pl.BlockSpec((_D_MODEL, _NUM_EXPERTS), lambda i: (0, 0)),
        ],
        out_specs=pl.BlockSpec((_NUM_EXPERTS, _BLOCK_ROWS), lambda i: (0, i)),
        out_shape=jax.ShapeDtypeStruct((_NUM_EXPERTS, n), jnp.float32),
    )(xf, wt)


def _make_sc_router(n):
    tw = n // _NW
    mesh = plsc.VectorSubcoreMesh(
        core_axis_name="c", subcore_axis_name="s",
        num_cores=_NUM_SC, num_subcores=_NUM_SUBCORES)

    @functools.partial(
        pl.kernel,
        mesh=mesh,
        out_type=[
            jax.ShapeDtypeStruct((n * _NUM_EXPERTS,), jnp.float32),
            jax.ShapeDtypeStruct((n * _TOP_K,), jnp.float32),
            jax.ShapeDtypeStruct((n * _TOP_K,), jnp.int32),
        ],
        scratch_types=(
            [pltpu.VMEM((_NUM_EXPERTS, tw), jnp.float32)]
            + [pltpu.VMEM((tw * _NUM_EXPERTS,), jnp.float32),
               pltpu.VMEM((tw * _TOP_K,), jnp.float32),
               pltpu.VMEM((tw * _TOP_K,), jnp.int32)]
        ),
        compiler_params=pltpu.CompilerParams(needs_layout_passes=False, use_tc_tiling_on_sc=True),
    )
    def sc_router(logits_hbm, probs_hbm, w_hbm, idx_hbm,
                  lbuf, pbuf, wbuf, ibuf):
        wid = lax.axis_index("s") * _NUM_SC + lax.axis_index("c")
        base = wid * tw
        pltpu.sync_copy(logits_hbm.at[:, pl.ds(base, tw)], lbuf)

        def step(j, carry):
            t0 = j * _LANES
            ls = [lbuf[e, pl.ds(t0, _LANES)] for e in range(_NUM_EXPERTS)]
            m = ls[0]
            for e in range(1, _NUM_EXPERTS):
                m = jnp.maximum(m, ls[e])
            es = [jnp.exp(l - m) for l in ls]
            s = es[0]
            for e in range(1, _NUM_EXPERTS):
                s = s + es[e]
            r = 1.0 / s

            iot = lax.iota(jnp.int32, _LANES)
            pb = t0 * _NUM_EXPERTS + iot * _NUM_EXPERTS
            for e in range(_NUM_EXPERTS):
                plsc.store_scatter(pbuf, [pb + e], es[e] * r)

            eight = jnp.full((_LANES,), _NUM_EXPERTS, jnp.int32)
            i1 = eight
            for e in range(_NUM_EXPERTS):
                cand = jnp.where(ls[e] == m,
                                 jnp.full((_LANES,), e, jnp.int32), eight)
                i1 = jnp.minimum(i1, cand)
            neg = jnp.full((_LANES,), -jnp.inf, jnp.float32)
            m2 = neg
            for e in range(_NUM_EXPERTS):
                le = jnp.where(i1 == e, neg, ls[e])
                m2 = jnp.maximum(m2, le)
            i2 = eight
            for e in range(_NUM_EXPERTS):
                le = jnp.where(i1 == e, neg, ls[e])
                cand = jnp.where(le == m2,
                                 jnp.full((_LANES,), e, jnp.int32), eight)
                i2 = jnp.minimum(i2, cand)

            p1 = r
            p2 = jnp.exp(m2 - m) * r
            ws = p1 + p2 + 1e-9
            wb = t0 * _TOP_K + iot * _TOP_K
            plsc.store_scatter(wbuf, [wb], p1 / ws)
            plsc.store_scatter(wbuf, [wb + 1], p2 / ws)
            plsc.store_scatter(ibuf, [wb], i1)
            plsc.store_scatter(ibuf, [wb + 1], i2)
            return carry

        lax.fori_loop(0, tw // _LANES, step, 0)

        pltpu.sync_copy(pbuf, probs_hbm.at[pl.ds(base * _NUM_EXPERTS,
                                                 tw * _NUM_EXPERTS)])
        pltpu.sync_copy(wbuf, w_hbm.at[pl.ds(base * _TOP_K, tw * _TOP_K)])
        pltpu.sync_copy(ibuf, idx_hbm.at[pl.ds(base * _TOP_K, tw * _TOP_K)])

    return sc_router


def kernel(x, W):
    B, S, D = x.shape
    n = B * S
    xf = x.reshape(n, D)
    wt = W.T

    logits_t = _gate_logits(xf, wt, n)
    probs_f, w_f, idx_f = _make_sc_router(n)(logits_t)

    return (w_f.reshape(B, S, _TOP_K),
            idx_f.reshape(B, S, _TOP_K),
            probs_f.reshape(B, S, _NUM_EXPERTS))


# speedup vs baseline: 3.6580x; 3.6580x over previous
import jax
import jax.numpy as jnp
from jax.experimental import pallas as pl

_D_MODEL = 768
_NUM_EXPERTS = 8
_TOP_K = 2
_BLOCK_ROWS = 4096


def _router_body(x_ref, wt_ref, probs_ref, w_ref, idx_ref):
    x_blk = x_ref[...]
    wt = wt_ref[...]
    logits = jnp.dot(x_blk, wt, preferred_element_type=jnp.float32)
    lt = logits.T

    m = jnp.max(lt, axis=0, keepdims=True)
    e = jnp.exp(lt - m)
    denom = jnp.sum(e, axis=0, keepdims=True)
    inv = 1.0 / denom

    iota = jax.lax.broadcasted_iota(jnp.int32, lt.shape, 0)
    i1 = jnp.min(jnp.where(lt == m, iota, _NUM_EXPERTS), axis=0,
                 keepdims=True)
    masked = jnp.where(iota == i1, -jnp.inf, lt)
    m2 = jnp.max(masked, axis=0, keepdims=True)
    i2 = jnp.min(jnp.where(masked == m2, iota, _NUM_EXPERTS), axis=0,
                 keepdims=True)

    p1 = inv
    p2 = jnp.exp(m2 - m) * inv
    wsum = p1 + p2 + 1e-9
    probs_ref[...] = e * inv
    w_ref[...] = jnp.concatenate([p1 / wsum, p2 / wsum], axis=0)
    idx_ref[...] = jnp.concatenate([i1, i2], axis=0)


def kernel(x, W):
    B, S, D = x.shape
    E = W.shape[0]
    n = B * S
    xf = x.reshape(n, D)
    wt = W.T

    grid = (n // _BLOCK_ROWS,)
    probs_t, weights_t, idx_t = pl.pallas_call(
        _router_body,
        grid=grid,
        in_specs=[
            pl.BlockSpec((_BLOCK_ROWS, D), lambda i: (i, 0)),
            pl.BlockSpec((D, E), lambda i: (0, 0)),
        ],
        out_specs=[
            pl.BlockSpec((E, _BLOCK_ROWS), lambda i: (0, i)),
            pl.BlockSpec((_TOP_K, _BLOCK_ROWS), lambda i: (0, i)),
            pl.BlockSpec((_TOP_K, _BLOCK_ROWS), lambda i: (0, i)),
        ],
        out_shape=[
            jax.ShapeDtypeStruct((E, n), jnp.float32),
            jax.ShapeDtypeStruct((_TOP_K, n), jnp.float32),
            jax.ShapeDtypeStruct((_TOP_K, n), jnp.int32),
        ],
    )(xf, wt)

    return (weights_t.T.reshape(B, S, _TOP_K),
            idx_t.T.reshape(B, S, _TOP_K),
            probs_t.T.reshape(B, S, E))


# speedup vs baseline: 3.6732x; 1.0042x over previous
import jax
import jax.numpy as jnp
from jax.experimental import pallas as pl

_D_MODEL = 768
_NUM_EXPERTS = 8
_TOP_K = 2
_BLOCK_ROWS = 4096


def _router_body(xa_ref, xb_ref, wt_ref, probs_ref, w_ref, idx_ref):
    wt = wt_ref[...]
    la = jnp.dot(xa_ref[...], wt, preferred_element_type=jnp.float32)
    lb = jnp.dot(xb_ref[...], wt, preferred_element_type=jnp.float32)
    logits = jnp.concatenate([la, lb], axis=0)
    lt = logits.T

    m = jnp.max(lt, axis=0, keepdims=True)
    e = jnp.exp(lt - m)
    denom = jnp.sum(e, axis=0, keepdims=True)
    inv = 1.0 / denom

    iota = jax.lax.broadcasted_iota(jnp.int32, lt.shape, 0)
    i1 = jnp.min(jnp.where(lt == m, iota, _NUM_EXPERTS), axis=0,
                 keepdims=True)
    masked = jnp.where(iota == i1, -jnp.inf, lt)
    m2 = jnp.max(masked, axis=0, keepdims=True)
    i2 = jnp.min(jnp.where(masked == m2, iota, _NUM_EXPERTS), axis=0,
                 keepdims=True)

    p1 = inv
    p2 = jnp.exp(m2 - m) * inv
    wsum = p1 + p2 + 1e-9
    probs_ref[...] = e * inv
    w_ref[...] = jnp.concatenate([p1 / wsum, p2 / wsum], axis=0)
    idx_ref[...] = jnp.concatenate([i1, i2], axis=0)


def kernel(x, W):
    B, S, D = x.shape
    E = W.shape[0]
    n = B * S
    xf = x.reshape(n, D)
    wt = W.T

    grid = (n // _BLOCK_ROWS,)
    probs_t, weights_t, idx_t = pl.pallas_call(
        _router_body,
        grid=grid,
        in_specs=[
            pl.BlockSpec((_BLOCK_ROWS // 2, D), lambda i: (2 * i, 0)),
            pl.BlockSpec((_BLOCK_ROWS // 2, D), lambda i: (2 * i + 1, 0)),
            pl.BlockSpec((D, E), lambda i: (0, 0)),
        ],
        out_specs=[
            pl.BlockSpec((E, _BLOCK_ROWS), lambda i: (0, i)),
            pl.BlockSpec((_TOP_K, _BLOCK_ROWS), lambda i: (0, i)),
            pl.BlockSpec((_TOP_K, _BLOCK_ROWS), lambda i: (0, i)),
        ],
        out_shape=[
            jax.ShapeDtypeStruct((E, n), jnp.float32),
            jax.ShapeDtypeStruct((_TOP_K, n), jnp.float32),
            jax.ShapeDtypeStruct((_TOP_K, n), jnp.int32),
        ],
    )(xf, xf, wt)

    return (weights_t.T.reshape(B, S, _TOP_K),
            idx_t.T.reshape(B, S, _TOP_K),
            probs_t.T.reshape(B, S, E))


# speedup vs baseline: 3.7000x; 1.0073x over previous
import jax
import jax.numpy as jnp
from jax.experimental import pallas as pl

_D_MODEL = 768
_NUM_EXPERTS = 8
_TOP_K = 2
_BLOCK_ROWS = 4096


def _router_body(x_ref, wt_ref, probs_ref, w_ref, idx_ref):
    x_blk = x_ref[...]
    wt = wt_ref[...]
    logits = jnp.dot(x_blk, wt, preferred_element_type=jnp.float32)
    lt = logits.T

    m = jnp.max(lt, axis=0, keepdims=True)
    e = jnp.exp(lt - m)
    denom = jnp.sum(e, axis=0, keepdims=True)
    inv = 1.0 / denom

    iota = jax.lax.broadcasted_iota(jnp.int32, lt.shape, 0)
    i1 = jnp.min(jnp.where(lt == m, iota, _NUM_EXPERTS), axis=0,
                 keepdims=True)
    masked = jnp.where(iota == i1, -jnp.inf, lt)
    m2 = jnp.max(masked, axis=0, keepdims=True)
    i2 = jnp.min(jnp.where(masked == m2, iota, _NUM_EXPERTS), axis=0,
                 keepdims=True)

    p1 = inv
    p2 = jnp.exp(m2 - m) * inv
    wsum = p1 + p2 + 1e-9
    probs_ref[...] = e * inv
    w_ref[...] = jnp.concatenate([p1 / wsum, p2 / wsum], axis=0)
    idx_ref[...] = jnp.concatenate([i1, i2], axis=0)


def kernel(x, W):
    B, S, D = x.shape
    E = W.shape[0]
    n = B * S
    xf = x.reshape(n, D)
    wt = W.T

    grid = (n // _BLOCK_ROWS,)
    probs_t, weights_t, idx_t = pl.pallas_call(
        _router_body,
        grid=grid,
        in_specs=[
            pl.BlockSpec((_BLOCK_ROWS, D), lambda i: (i, 0)),
            pl.BlockSpec((D, E), lambda i: (0, 0)),
        ],
        out_specs=[
            pl.BlockSpec((E, _BLOCK_ROWS), lambda i: (0, i)),
            pl.BlockSpec((_TOP_K, _BLOCK_ROWS), lambda i: (0, i)),
            pl.BlockSpec((_TOP_K, _BLOCK_ROWS), lambda i: (0, i)),
        ],
        out_shape=[
            jax.ShapeDtypeStruct((E, n), jnp.float32),
            jax.ShapeDtypeStruct((_TOP_K, n), jnp.float32),
            jax.ShapeDtypeStruct((_TOP_K, n), jnp.int32),
        ],
    )(xf, wt)

    return (weights_t.T.reshape(B, S, _TOP_K),
            idx_t.T.reshape(B, S, _TOP_K),
            probs_t.T.reshape(B, S, E))
